# baseline (speedup 1.0000x reference)
import functools

import jax
import jax.numpy as jnp
from jax import lax
from jax.experimental import pallas as pl
from jax.experimental.pallas import tpu as pltpu

B = 2
S = 2048
NUM_OPS = 64
NUM_BIND = 128
SIG_DIM = 256
RULE_DIM = 256
UB = NUM_OPS * NUM_BIND

TILE_T = 128


def _body(qu_ref, qb_ref, qs_ref, usage_ref, ema_ref, pd_ref, ps_ref,
          delta_ref, sig_ref, conf_ref, w4_ref, w_ref, *, precision):
    qb = qb_ref[...]
    qu = qu_ref[...]
    mask = (usage_ref[...] > 0.0).astype(jnp.float32)

    for u in range(NUM_OPS):
        w_ref[:, u * NUM_BIND:(u + 1) * NUM_BIND] = (
            qu_ref[:, u:u + 1] * qb * mask[u:u + 1, :])

    cat = jnp.concatenate([mask, ema_ref[...] * mask], axis=1)
    r = lax.dot_general(qu, cat, (((1,), (0,)), ((), ())),
                        precision=lax.Precision.HIGHEST,
                        preferred_element_type=jnp.float32)
    denom = jnp.sum(qb * r[:, :NUM_BIND], axis=1, keepdims=True)
    base = jnp.sum(qb * r[:, NUM_BIND:], axis=1, keepdims=True)
    scale = jnp.where(denom > 0.0,
                      1.0 / jnp.maximum(denom, 1e-6),
                      jnp.zeros_like(denom))

    w = w_ref[...]
    d = lax.dot_general(w, pd_ref[...], (((1,), (0,)), ((), ())),
                        precision=precision,
                        preferred_element_type=jnp.float32)
    s = lax.dot_general(w, ps_ref[...], (((1,), (0,)), ((), ())),
                        precision=precision,
                        preferred_element_type=jnp.float32)

    d = d * scale
    s = s * scale
    base = base * scale
    delta_ref[...] = d
    sig_ref[...] = s
    w4_ref[0] = (w * scale).reshape(w.shape[0], NUM_OPS, NUM_BIND)

    qs = qs_ref[...]
    qs_norm = jnp.sqrt(jnp.sum(qs * qs, axis=1, keepdims=True))
    qsn = qs / jnp.maximum(qs_norm, 1e-12)
    ms = s + 1e-6
    ms_norm = jnp.sqrt(jnp.sum(ms * ms, axis=1, keepdims=True))
    msn = ms / jnp.maximum(ms_norm, 1e-12)
    qsn_n = jnp.sqrt(jnp.sum(qsn * qsn, axis=1, keepdims=True))
    msn_n = jnp.sqrt(jnp.sum(msn * msn, axis=1, keepdims=True))
    cos = (jnp.sum(qsn * msn, axis=1, keepdims=True)
           / jnp.maximum(qsn_n * msn_n, 1e-8))
    conf = jnp.clip(base * 0.5 * (1.0 + cos), 0.0, 1.0)
    conf_ref[...] = jnp.broadcast_to(conf, (conf.shape[0], 8))


def _retrieve(q_u, q_b, q_sigma, delta_rule_proto, signature_proto,
              usage_count, ema_conf, precision=lax.Precision.DEFAULT,
              interpret=False):
    T = B * S
    qu2 = q_u.reshape(T, NUM_OPS)
    qb2 = q_b.reshape(T, NUM_BIND)
    qs2 = q_sigma.reshape(T, SIG_DIM)
    pd2 = delta_rule_proto.reshape(UB, RULE_DIM)
    ps2 = signature_proto.reshape(UB, SIG_DIM)

    grid = (T // TILE_T,)
    tok = lambda i: (i, 0)
    rep = lambda i: (0, 0)
    out = pl.pallas_call(
        functools.partial(_body, precision=precision),
        grid=grid,
        in_specs=[
            pl.BlockSpec((TILE_T, NUM_OPS), tok),
            pl.BlockSpec((TILE_T, NUM_BIND), tok),
            pl.BlockSpec((TILE_T, SIG_DIM), tok),
            pl.BlockSpec((NUM_OPS, NUM_BIND), rep),
            pl.BlockSpec((NUM_OPS, NUM_BIND), rep),
            pl.BlockSpec((UB, RULE_DIM), rep),
            pl.BlockSpec((UB, SIG_DIM), rep),
        ],
        out_specs=[
            pl.BlockSpec((TILE_T, RULE_DIM), tok),
            pl.BlockSpec((TILE_T, SIG_DIM), tok),
            pl.BlockSpec((TILE_T, 8), tok),
            pl.BlockSpec((1, TILE_T, NUM_OPS, NUM_BIND),
                         lambda i: (i // (S // TILE_T), i % (S // TILE_T),
                                    0, 0)),
        ],
        out_shape=[
            jax.ShapeDtypeStruct((T, RULE_DIM), jnp.float32),
            jax.ShapeDtypeStruct((T, SIG_DIM), jnp.float32),
            jax.ShapeDtypeStruct((T, 8), jnp.float32),
            jax.ShapeDtypeStruct((B, S, NUM_OPS, NUM_BIND), jnp.float32),
        ],
        scratch_shapes=[pltpu.VMEM((TILE_T, UB), jnp.float32)],
        compiler_params=pltpu.CompilerParams(
            dimension_semantics=("parallel",)),
        interpret=interpret,
    )(qu2, qb2, qs2, usage_count, ema_conf, pd2, ps2)
    d, s, c, w = out
    return (d.reshape(B, S, RULE_DIM),
            s.reshape(B, S, SIG_DIM),
            c[:, :1].reshape(B, S, 1),
            w)


def kernel(q_u, q_b, q_sigma, delta_rule_proto, signature_proto,
           usage_count, ema_conf):
    return _retrieve(q_u, q_b, q_sigma, delta_rule_proto, signature_proto,
                     usage_count, ema_conf)


# speedup vs baseline: 1.1305x; 1.1305x over previous
import functools

import jax
import jax.numpy as jnp
from jax import lax
from jax.experimental import pallas as pl
from jax.experimental.pallas import tpu as pltpu

B = 2
S = 2048
NUM_OPS = 64
NUM_BIND = 128
SIG_DIM = 256
RULE_DIM = 256
UB = NUM_OPS * NUM_BIND

TILE_T = 256


def _body(qu_ref, qb_ref, qs_ref, usage_ref, ema_ref, pd_ref, ps_ref,
          delta_ref, sig_ref, conf_ref, w4_ref, w_ref, *, precision):
    qb = qb_ref[...]
    qu = qu_ref[...]
    mask = (usage_ref[...] > 0.0).astype(jnp.float32)

    for u in range(NUM_OPS):
        w_ref[:, u * NUM_BIND:(u + 1) * NUM_BIND] = (
            qu_ref[:, u:u + 1] * qb * mask[u:u + 1, :])

    cat = jnp.concatenate([mask, ema_ref[...] * mask], axis=1)
    r = lax.dot_general(qu, cat, (((1,), (0,)), ((), ())),
                        precision=lax.Precision.HIGHEST,
                        preferred_element_type=jnp.float32)
    denom = jnp.sum(qb * r[:, :NUM_BIND], axis=1, keepdims=True)
    base = jnp.sum(qb * r[:, NUM_BIND:], axis=1, keepdims=True)
    scale = jnp.where(denom > 0.0,
                      1.0 / jnp.maximum(denom, 1e-6),
                      jnp.zeros_like(denom))

    w = w_ref[...]
    d = lax.dot_general(w, pd_ref[...], (((1,), (0,)), ((), ())),
                        precision=precision,
                        preferred_element_type=jnp.float32)
    s = lax.dot_general(w, ps_ref[...], (((1,), (0,)), ((), ())),
                        precision=precision,
                        preferred_element_type=jnp.float32)

    d = d * scale
    s = s * scale
    base = base * scale
    delta_ref[...] = d
    sig_ref[...] = s
    w4_ref[0] = (w * scale).reshape(w.shape[0], NUM_OPS, NUM_BIND)

    qs = qs_ref[...]
    qs_norm = jnp.sqrt(jnp.sum(qs * qs, axis=1, keepdims=True))
    qsn = qs / jnp.maximum(qs_norm, 1e-12)
    ms = s + 1e-6
    ms_norm = jnp.sqrt(jnp.sum(ms * ms, axis=1, keepdims=True))
    msn = ms / jnp.maximum(ms_norm, 1e-12)
    qsn_n = jnp.sqrt(jnp.sum(qsn * qsn, axis=1, keepdims=True))
    msn_n = jnp.sqrt(jnp.sum(msn * msn, axis=1, keepdims=True))
    cos = (jnp.sum(qsn * msn, axis=1, keepdims=True)
           / jnp.maximum(qsn_n * msn_n, 1e-8))
    conf = jnp.clip(base * 0.5 * (1.0 + cos), 0.0, 1.0)
    conf_ref[...] = jnp.broadcast_to(conf, (conf.shape[0], 8))


def _retrieve(q_u, q_b, q_sigma, delta_rule_proto, signature_proto,
              usage_count, ema_conf, precision=lax.Precision.DEFAULT,
              interpret=False):
    T = B * S
    qu2 = q_u.reshape(T, NUM_OPS)
    qb2 = q_b.reshape(T, NUM_BIND)
    qs2 = q_sigma.reshape(T, SIG_DIM)
    pd2 = delta_rule_proto.reshape(UB, RULE_DIM)
    ps2 = signature_proto.reshape(UB, SIG_DIM)

    grid = (T // TILE_T,)
    tok = lambda i: (i, 0)
    rep = lambda i: (0, 0)
    out = pl.pallas_call(
        functools.partial(_body, precision=precision),
        grid=grid,
        in_specs=[
            pl.BlockSpec((TILE_T, NUM_OPS), tok),
            pl.BlockSpec((TILE_T, NUM_BIND), tok),
            pl.BlockSpec((TILE_T, SIG_DIM), tok),
            pl.BlockSpec((NUM_OPS, NUM_BIND), rep),
            pl.BlockSpec((NUM_OPS, NUM_BIND), rep),
            pl.BlockSpec((UB, RULE_DIM), rep),
            pl.BlockSpec((UB, SIG_DIM), rep),
        ],
        out_specs=[
            pl.BlockSpec((TILE_T, RULE_DIM), tok),
            pl.BlockSpec((TILE_T, SIG_DIM), tok),
            pl.BlockSpec((TILE_T, 8), tok),
            pl.BlockSpec((1, TILE_T, NUM_OPS, NUM_BIND),
                         lambda i: (i // (S // TILE_T), i % (S // TILE_T),
                                    0, 0)),
        ],
        out_shape=[
            jax.ShapeDtypeStruct((T, RULE_DIM), jnp.float32),
            jax.ShapeDtypeStruct((T, SIG_DIM), jnp.float32),
            jax.ShapeDtypeStruct((T, 8), jnp.float32),
            jax.ShapeDtypeStruct((B, S, NUM_OPS, NUM_BIND), jnp.float32),
        ],
        scratch_shapes=[pltpu.VMEM((TILE_T, UB), jnp.float32)],
        compiler_params=pltpu.CompilerParams(
            dimension_semantics=("parallel",)),
        interpret=interpret,
    )(qu2, qb2, qs2, usage_count, ema_conf, pd2, ps2)
    d, s, c, w = out
    return (d.reshape(B, S, RULE_DIM),
            s.reshape(B, S, SIG_DIM),
            c[:, :1].reshape(B, S, 1),
            w)


def kernel(q_u, q_b, q_sigma, delta_rule_proto, signature_proto,
           usage_count, ema_conf):
    return _retrieve(q_u, q_b, q_sigma, delta_rule_proto, signature_proto,
                     usage_count, ema_conf)
